# baseline (device time: 193115 ns/iter reference)
import os

import jax
import jax.numpy as jnp
from jax import lax
from jax.experimental import pallas as pl
from jax.experimental.pallas import tpu as pltpu

N_DEV = 16
SQ = 2048
D_MODEL = 1024
H_LOCAL = 8
DH = 128
HD_LOCAL = H_LOCAL * DH
CHUNK = SQ // N_DEV
QT = 512
N_QT = SQ // QT
CPT = QT // CHUNK
BLK = 64
SCALE = 0.08838834764831843
_NO_RING = os.environ.get("NO_RING") == "1"


def kernel(x, Wq, K_ext, V_ext, Wo):
    my = lax.axis_index("i")
    x2 = x.reshape(SQ, D_MODEL).astype(jnp.bfloat16)
    k2 = K_ext.reshape(SQ, HD_LOCAL).astype(jnp.bfloat16)
    v2 = V_ext.reshape(SQ, HD_LOCAL)
    wq_l = lax.dynamic_slice_in_dim(Wq, my * HD_LOCAL, HD_LOCAL, axis=1)
    wo_l = lax.dynamic_slice_in_dim(Wo, my * HD_LOCAL, HD_LOCAL, axis=0)
    wq_l = wq_l.astype(jnp.bfloat16)
    wo_l = wo_l.astype(jnp.bfloat16)

    def body(x_ref, wq_ref, k_ref, v_ref, wo_ref, out_ref,
             rs_ref, sb_ref,
             rs_send, rs_recv, ag_send, ag_recv):
        me = lax.axis_index("i")

        barrier = pltpu.get_barrier_semaphore()
        for d in range(1, N_DEV):
            pl.semaphore_signal(barrier, inc=1,
                                device_id=(lax.rem(me + d, N_DEV),),
                                device_id_type=pl.DeviceIdType.MESH)
        pl.semaphore_wait(barrier, N_DEV - 1)

        def chunk_sl(c):
            return (pl.ds(c * CHUNK, CHUNK), slice(None))

        def rs_rdma(c, d):
            return pltpu.make_async_remote_copy(
                src_ref=sb_ref.at[chunk_sl(c)],
                dst_ref=rs_ref.at[d],
                send_sem=rs_send.at[d], recv_sem=rs_recv.at[d],
                device_id=(c,), device_id_type=pl.DeviceIdType.MESH)

        rb = lax.broadcasted_iota(jnp.int32, (QT, QT), 0) // BLK
        cb = lax.broadcasted_iota(jnp.int32, (QT, QT), 1) // BLK
        diag_mask = rb >= cb
        for t in range(N_QT):
            q_t = jnp.dot(x_ref[t * QT:(t + 1) * QT, :], wq_ref[...],
                          preferred_element_type=jnp.float32
                          ).astype(jnp.bfloat16)
            po = jnp.zeros((QT, D_MODEL), jnp.float32)
            for h in range(H_LOCAL):
                qh = q_t[:, h * DH:(h + 1) * DH]
                kd = k_ref[t * QT:(t + 1) * QT, h * DH:(h + 1) * DH]
                sd = lax.dot_general(qh, kd, (((1,), (1,)), ((), ())),
                                     preferred_element_type=jnp.float32)
                ed = jnp.where(diag_mask, jnp.exp(sd * SCALE), 0.0)
                vd = v_ref[t * QT:(t + 1) * QT, h * DH:(h + 1) * DH]
                if t > 0:
                    kf = k_ref[0:t * QT, h * DH:(h + 1) * DH]
                    sf = lax.dot_general(qh, kf, (((1,), (1,)), ((), ())),
                                         preferred_element_type=jnp.float32)
                    ef = jnp.exp(sf * SCALE)
                    d = (jnp.sum(ef, axis=1, keepdims=True)
                         + jnp.sum(ed, axis=1, keepdims=True))
                    ctx = (jnp.dot(ef, v_ref[0:t * QT, h * DH:(h + 1) * DH],
                                   preferred_element_type=jnp.float32)
                           + jnp.dot(ed, vd,
                                     preferred_element_type=jnp.float32))
                else:
                    d = jnp.sum(ed, axis=1, keepdims=True)
                    ctx = jnp.dot(ed, vd, preferred_element_type=jnp.float32)
                ctx = (ctx * (1.0 / d)).astype(jnp.bfloat16)
                po = po + jnp.dot(ctx, wo_ref[h * DH:(h + 1) * DH, :],
                                  preferred_element_type=jnp.float32)
            out_ref[t * QT:(t + 1) * QT, :] = po
            if not _NO_RING:
                for j in range(CPT):
                    c = t * CPT + j
                    sb_ref[chunk_sl(c)] = (
                        out_ref[chunk_sl(c)].astype(jnp.bfloat16))
                    doff = lax.rem(me + 2 * N_DEV - c, N_DEV)
                    rdma = rs_rdma(c, doff)

                    @pl.when(doff != 0)
                    def _():
                        rdma.start()

        if not _NO_RING:
            for d in range(1, N_DEV):
                rs_rdma(0, d).wait_recv()
            red = out_ref[pl.ds(me * CHUNK, CHUNK), :]
            for d in range(1, N_DEV):
                red = red + rs_ref[d].astype(jnp.float32)
            sb_ref[pl.ds(me * CHUNK, CHUNK), :] = red.astype(jnp.bfloat16)

            for d in range(1, N_DEV):
                c = lax.rem(me + 2 * N_DEV - d, N_DEV)
                pltpu.make_async_remote_copy(
                    src_ref=sb_ref.at[(pl.ds(c * CHUNK, CHUNK),
                                       slice(None))],
                    dst_ref=rs_ref.at[d],
                    send_sem=rs_send.at[d], recv_sem=rs_recv.at[d],
                    device_id=(0,), device_id_type=pl.DeviceIdType.MESH,
                ).wait_send()

            my_sl = (pl.ds(me * CHUNK, CHUNK), slice(None))
            dummy_sl = (pl.ds(0, CHUNK), slice(None))
            for d in range(1, N_DEV):
                pltpu.make_async_remote_copy(
                    src_ref=sb_ref.at[my_sl], dst_ref=sb_ref.at[my_sl],
                    send_sem=ag_send.at[d], recv_sem=ag_recv.at[d],
                    device_id=(lax.rem(me + d, N_DEV),),
                    device_id_type=pl.DeviceIdType.MESH,
                ).start()
            for d in range(1, N_DEV):
                pltpu.make_async_remote_copy(
                    src_ref=sb_ref.at[dummy_sl], dst_ref=sb_ref.at[dummy_sl],
                    send_sem=ag_send.at[d], recv_sem=ag_recv.at[d],
                    device_id=(0,), device_id_type=pl.DeviceIdType.MESH,
                ).wait_recv()
            out_ref[...] = sb_ref[...].astype(jnp.float32)
            for d in range(1, N_DEV):
                pltpu.make_async_remote_copy(
                    src_ref=sb_ref.at[my_sl], dst_ref=sb_ref.at[my_sl],
                    send_sem=ag_send.at[d], recv_sem=ag_recv.at[d],
                    device_id=(0,), device_id_type=pl.DeviceIdType.MESH,
                ).wait_send()

        def _exit(second_barrier):
            for d in range(1, N_DEV):
                pl.semaphore_signal(second_barrier, inc=1,
                                    device_id=(lax.rem(me + d, N_DEV),),
                                    device_id_type=pl.DeviceIdType.MESH)
            pl.semaphore_wait(second_barrier, N_DEV - 1)
        pl.run_scoped(_exit, second_barrier=pltpu.SemaphoreType.REGULAR)

    out = pl.pallas_call(
        body,
        out_shape=jax.ShapeDtypeStruct((SQ, D_MODEL), jnp.float32),
        in_specs=[pl.BlockSpec(memory_space=pltpu.VMEM)] * 5,
        out_specs=pl.BlockSpec(memory_space=pltpu.VMEM),
        scratch_shapes=[
            pltpu.VMEM((N_DEV, CHUNK, D_MODEL), jnp.bfloat16),
            pltpu.VMEM((SQ, D_MODEL), jnp.bfloat16),
            pltpu.SemaphoreType.DMA((N_DEV,)),
            pltpu.SemaphoreType.DMA((N_DEV,)),
            pltpu.SemaphoreType.DMA((N_DEV,)),
            pltpu.SemaphoreType.DMA((N_DEV,)),
        ],
        compiler_params=pltpu.CompilerParams(
            collective_id=0, vmem_limit_bytes=100 * 1024 * 1024),
    )(x2, wq_l, k2, v2, wo_l)
    return out.reshape(1, SQ, D_MODEL)


# device time: 153428 ns/iter; 1.2587x vs baseline; 1.2587x over previous
import os

import jax
import jax.numpy as jnp
from jax import lax
from jax.experimental import pallas as pl
from jax.experimental.pallas import tpu as pltpu

N_DEV = 16
N_PLANE = 4
N_Z = 4
SQ = 2048
D_MODEL = 1024
H_LOCAL = 8
DH = 128
HD_LOCAL = H_LOCAL * DH
CHUNK = SQ // N_DEV
QT = 512
N_QT = SQ // QT
CPT = QT // CHUNK
BLK = 64
SCALE = 0.08838834764831843
_NO_RING = os.environ.get("NO_RING") == "1"


def kernel(x, Wq, K_ext, V_ext, Wo):
    my = lax.axis_index("i")
    x2 = x.reshape(SQ, D_MODEL).astype(jnp.bfloat16)
    k2 = K_ext.reshape(SQ, HD_LOCAL).astype(jnp.bfloat16)
    v2 = V_ext.reshape(SQ, HD_LOCAL)
    wq_l = lax.dynamic_slice_in_dim(Wq, my * HD_LOCAL, HD_LOCAL, axis=1)
    wo_l = lax.dynamic_slice_in_dim(Wo, my * HD_LOCAL, HD_LOCAL, axis=0)
    wq_l = wq_l.astype(jnp.bfloat16)
    wo_l = wo_l.astype(jnp.bfloat16)

    def body(x_ref, wq_ref, k_ref, v_ref, wo_ref, out_ref,
             rs1_ref, rs2_ref, sb_ref, acc_ref,
             rs1_send, rs1_recv, rs2_send, rs2_recv,
             aga_send, aga_recv, agb_send, agb_recv):
        me = lax.axis_index("i")
        p_me = lax.rem(me, N_PLANE)
        z_me = me // N_PLANE
        plane_base = me - p_me

        def plane_peer(dp):
            return plane_base + lax.rem(me + dp, N_PLANE)

        def z_peer(dz):
            return lax.rem(me + N_PLANE * dz, N_DEV)

        peers = [plane_peer(dp) for dp in range(1, N_PLANE)] + [
            z_peer(dz) for dz in range(1, N_Z)]

        barrier = pltpu.get_barrier_semaphore()
        for peer in peers:
            pl.semaphore_signal(barrier, inc=1, device_id=(peer,),
                                device_id_type=pl.DeviceIdType.MESH)
        pl.semaphore_wait(barrier, len(peers))

        def chunk_sl(c):
            return (pl.ds(c * CHUNK, CHUNK), slice(None))

        def rs1_rdma(c):
            dp = lax.rem(me - c + 2 * N_DEV, N_PLANE)
            slot = jnp.maximum(dp, 1) - 1
            idx = (c // N_PLANE) * 3 + slot
            return pltpu.make_async_remote_copy(
                src_ref=sb_ref.at[chunk_sl(c)],
                dst_ref=rs1_ref.at[c // N_PLANE, slot],
                send_sem=rs1_send.at[idx], recv_sem=rs1_recv.at[idx],
                device_id=(plane_base + (c % N_PLANE),),
                device_id_type=pl.DeviceIdType.MESH)

        rb = lax.broadcasted_iota(jnp.int32, (QT, QT), 0) // BLK
        cb = lax.broadcasted_iota(jnp.int32, (QT, QT), 1) // BLK
        diag_mask = rb >= cb
        for t in range(N_QT):
            q_t = jnp.dot(x_ref[t * QT:(t + 1) * QT, :], wq_ref[...],
                          preferred_element_type=jnp.float32
                          ).astype(jnp.bfloat16)
            po = jnp.zeros((QT, D_MODEL), jnp.float32)
            for h in range(H_LOCAL):
                qh = q_t[:, h * DH:(h + 1) * DH]
                kd = k_ref[t * QT:(t + 1) * QT, h * DH:(h + 1) * DH]
                sd = lax.dot_general(qh, kd, (((1,), (1,)), ((), ())),
                                     preferred_element_type=jnp.float32)
                ed = jnp.where(diag_mask, jnp.exp(sd * SCALE), 0.0)
                vd = v_ref[t * QT:(t + 1) * QT, h * DH:(h + 1) * DH]
                if t > 0:
                    kf = k_ref[0:t * QT, h * DH:(h + 1) * DH]
                    sf = lax.dot_general(qh, kf, (((1,), (1,)), ((), ())),
                                         preferred_element_type=jnp.float32)
                    ef = jnp.exp(sf * SCALE)
                    d = (jnp.sum(ef, axis=1, keepdims=True)
                         + jnp.sum(ed, axis=1, keepdims=True))
                    ctx = (jnp.dot(ef, v_ref[0:t * QT, h * DH:(h + 1) * DH],
                                   preferred_element_type=jnp.float32)
                           + jnp.dot(ed, vd,
                                     preferred_element_type=jnp.float32))
                else:
                    d = jnp.sum(ed, axis=1, keepdims=True)
                    ctx = jnp.dot(ed, vd, preferred_element_type=jnp.float32)
                ctx = (ctx * (1.0 / d)).astype(jnp.bfloat16)
                po = po + jnp.dot(ctx, wo_ref[h * DH:(h + 1) * DH, :],
                                  preferred_element_type=jnp.float32)
            out_ref[t * QT:(t + 1) * QT, :] = po
            if not _NO_RING:
                for j in range(CPT):
                    c = t * CPT + j
                    sb_ref[chunk_sl(c)] = (
                        out_ref[chunk_sl(c)].astype(jnp.bfloat16))
                    rdma = rs1_rdma(c)

                    @pl.when(lax.rem(me, N_PLANE) != c % N_PLANE)
                    def _():
                        rdma.start()

        if not _NO_RING:
            for idx in range(12):
                pltpu.make_async_remote_copy(
                    src_ref=sb_ref.at[chunk_sl(0)],
                    dst_ref=rs1_ref.at[idx // 3, idx % 3],
                    send_sem=rs1_send.at[idx], recv_sem=rs1_recv.at[idx],
                    device_id=(0,), device_id_type=pl.DeviceIdType.MESH,
                ).wait_recv()
            for j in range(N_Z):
                cj = N_PLANE * j + p_me
                agg = out_ref[pl.ds(cj * CHUNK, CHUNK), :]
                for k in range(3):
                    agg = agg + rs1_ref[j, k].astype(jnp.float32)
                @pl.when(j == z_me)
                def _():
                    acc_ref[...] = agg
                sb_ref[pl.ds(cj * CHUNK, CHUNK), :] = agg.astype(jnp.bfloat16)
                dz = lax.rem(z_me - j + N_Z, N_Z)
                zslot = jnp.maximum(dz, 1) - 1
                rdma2 = pltpu.make_async_remote_copy(
                    src_ref=sb_ref.at[(pl.ds(cj * CHUNK, CHUNK),
                                       slice(None))],
                    dst_ref=rs2_ref.at[zslot],
                    send_sem=rs2_send.at[zslot], recv_sem=rs2_recv.at[zslot],
                    device_id=(N_PLANE * j + p_me,),
                    device_id_type=pl.DeviceIdType.MESH)

                @pl.when(j != z_me)
                def _():
                    rdma2.start()

            my_sl = (pl.ds(me * CHUNK, CHUNK), slice(None))
            dummy_sl = (pl.ds(0, CHUNK), slice(None))

            def dummy_rdma(send_sems, recv_sems, k):
                return pltpu.make_async_remote_copy(
                    src_ref=sb_ref.at[dummy_sl], dst_ref=rs2_ref.at[k % 3],
                    send_sem=send_sems.at[k], recv_sem=recv_sems.at[k],
                    device_id=(0,), device_id_type=pl.DeviceIdType.MESH)

            for k in range(3):
                dummy_rdma(rs2_send, rs2_recv, k).wait_recv()
            red = acc_ref[...]
            for k in range(3):
                red = red + rs2_ref[k].astype(jnp.float32)
            sb_ref[my_sl] = red.astype(jnp.bfloat16)

            for idx in range(12):
                pltpu.make_async_remote_copy(
                    src_ref=sb_ref.at[dummy_sl],
                    dst_ref=rs1_ref.at[idx // 3, idx % 3],
                    send_sem=rs1_send.at[idx], recv_sem=rs1_recv.at[idx],
                    device_id=(0,), device_id_type=pl.DeviceIdType.MESH,
                ).wait_send()
            for k in range(3):
                dummy_rdma(rs2_send, rs2_recv, k).wait_send()

            for dz in range(1, N_Z):
                pltpu.make_async_remote_copy(
                    src_ref=sb_ref.at[my_sl], dst_ref=sb_ref.at[my_sl],
                    send_sem=aga_send.at[dz - 1], recv_sem=aga_recv.at[dz - 1],
                    device_id=(z_peer(dz),),
                    device_id_type=pl.DeviceIdType.MESH,
                ).start()
            for k in range(3):
                pltpu.make_async_remote_copy(
                    src_ref=sb_ref.at[dummy_sl], dst_ref=sb_ref.at[dummy_sl],
                    send_sem=aga_send.at[k], recv_sem=aga_recv.at[k],
                    device_id=(0,), device_id_type=pl.DeviceIdType.MESH,
                ).wait_recv()

            for j in range(N_Z):
                cj_sl = (pl.ds((N_PLANE * j + p_me) * CHUNK, CHUNK),
                         slice(None))
                for dp in range(1, N_PLANE):
                    pltpu.make_async_remote_copy(
                        src_ref=sb_ref.at[cj_sl], dst_ref=sb_ref.at[cj_sl],
                        send_sem=agb_send.at[j * 3 + dp - 1],
                        recv_sem=agb_recv.at[j * 3 + dp - 1],
                        device_id=(plane_peer(dp),),
                        device_id_type=pl.DeviceIdType.MESH,
                    ).start()
            for idx in range(12):
                pltpu.make_async_remote_copy(
                    src_ref=sb_ref.at[dummy_sl], dst_ref=sb_ref.at[dummy_sl],
                    send_sem=agb_send.at[idx], recv_sem=agb_recv.at[idx],
                    device_id=(0,), device_id_type=pl.DeviceIdType.MESH,
                ).wait_recv()

            out_ref[...] = sb_ref[...].astype(jnp.float32)

            for k in range(3):
                pltpu.make_async_remote_copy(
                    src_ref=sb_ref.at[my_sl], dst_ref=sb_ref.at[my_sl],
                    send_sem=aga_send.at[k], recv_sem=aga_recv.at[k],
                    device_id=(0,), device_id_type=pl.DeviceIdType.MESH,
                ).wait_send()
            for idx in range(12):
                pltpu.make_async_remote_copy(
                    src_ref=sb_ref.at[dummy_sl], dst_ref=sb_ref.at[dummy_sl],
                    send_sem=agb_send.at[idx], recv_sem=agb_recv.at[idx],
                    device_id=(0,), device_id_type=pl.DeviceIdType.MESH,
                ).wait_send()

        def _exit(second_barrier):
            for peer in peers:
                pl.semaphore_signal(second_barrier, inc=1, device_id=(peer,),
                                    device_id_type=pl.DeviceIdType.MESH)
            pl.semaphore_wait(second_barrier, len(peers))
        pl.run_scoped(_exit, second_barrier=pltpu.SemaphoreType.REGULAR)

    out = pl.pallas_call(
        body,
        out_shape=jax.ShapeDtypeStruct((SQ, D_MODEL), jnp.float32),
        in_specs=[pl.BlockSpec(memory_space=pltpu.VMEM)] * 5,
        out_specs=pl.BlockSpec(memory_space=pltpu.VMEM),
        scratch_shapes=[
            pltpu.VMEM((N_Z, 3, CHUNK, D_MODEL), jnp.bfloat16),
            pltpu.VMEM((3, CHUNK, D_MODEL), jnp.bfloat16),
            pltpu.VMEM((SQ, D_MODEL), jnp.bfloat16),
            pltpu.VMEM((CHUNK, D_MODEL), jnp.float32),
            pltpu.SemaphoreType.DMA((12,)),
            pltpu.SemaphoreType.DMA((12,)),
            pltpu.SemaphoreType.DMA((3,)),
            pltpu.SemaphoreType.DMA((3,)),
            pltpu.SemaphoreType.DMA((3,)),
            pltpu.SemaphoreType.DMA((3,)),
            pltpu.SemaphoreType.DMA((12,)),
            pltpu.SemaphoreType.DMA((12,)),
        ],
        compiler_params=pltpu.CompilerParams(
            collective_id=0, vmem_limit_bytes=100 * 1024 * 1024),
    )(x2, wq_l, k2, v2, wo_l)
    return out.reshape(1, SQ, D_MODEL)


# device time: 141148 ns/iter; 1.3682x vs baseline; 1.0870x over previous
import os

import jax
import jax.numpy as jnp
from jax import lax
from jax.experimental import pallas as pl
from jax.experimental.pallas import tpu as pltpu

N_DEV = 16
N_PLANE = 4
N_Z = 4
SQ = 2048
D_MODEL = 1024
H_LOCAL = 8
DH = 128
HD_LOCAL = H_LOCAL * DH
CHUNK = SQ // N_DEV
QT = 512
N_QT = SQ // QT
CPT = QT // CHUNK
BLK = 64
SCALE = 0.08838834764831843
_NO_RING = os.environ.get("NO_RING") == "1"


def kernel(x, Wq, K_ext, V_ext, Wo):
    my = lax.axis_index("i")
    x2 = x.reshape(SQ, D_MODEL).astype(jnp.bfloat16)
    k2 = K_ext.reshape(SQ, HD_LOCAL).astype(jnp.bfloat16)
    v2 = V_ext.reshape(SQ, HD_LOCAL)
    wq_l = lax.dynamic_slice_in_dim(Wq, my * HD_LOCAL, HD_LOCAL, axis=1)
    wo_l = lax.dynamic_slice_in_dim(Wo, my * HD_LOCAL, HD_LOCAL, axis=0)
    wq_l = wq_l.astype(jnp.bfloat16)
    wo_l = wo_l.astype(jnp.bfloat16)

    def body(x_ref, wq_ref, k_ref, v_ref, wo_ref, out_ref,
             rs1_ref, rs2_ref, sb_ref, acc_ref, ctx_ref,
             rs1_send, rs1_recv, rs2_send, rs2_recv,
             aga_send, aga_recv, agb_send, agb_recv):
        me = lax.axis_index("i")
        p_me = lax.rem(me, N_PLANE)
        z_me = me // N_PLANE
        plane_base = me - p_me

        def plane_peer(dp):
            return plane_base + lax.rem(me + dp, N_PLANE)

        def z_peer(dz):
            return lax.rem(me + N_PLANE * dz, N_DEV)

        peers = [plane_peer(dp) for dp in range(1, N_PLANE)] + [
            z_peer(dz) for dz in range(1, N_Z)]

        barrier = pltpu.get_barrier_semaphore()
        for peer in peers:
            pl.semaphore_signal(barrier, inc=1, device_id=(peer,),
                                device_id_type=pl.DeviceIdType.MESH)
        pl.semaphore_wait(barrier, len(peers))

        def chunk_sl(c):
            return (pl.ds(c * CHUNK, CHUNK), slice(None))

        def rs1_rdma(c):
            dp = lax.rem(me - c + 2 * N_DEV, N_PLANE)
            slot = jnp.maximum(dp, 1) - 1
            idx = (c // N_PLANE) * 3 + slot
            return pltpu.make_async_remote_copy(
                src_ref=sb_ref.at[chunk_sl(c)],
                dst_ref=rs1_ref.at[c // N_PLANE, slot],
                send_sem=rs1_send.at[idx], recv_sem=rs1_recv.at[idx],
                device_id=(plane_base + (c % N_PLANE),),
                device_id_type=pl.DeviceIdType.MESH)

        rb = lax.broadcasted_iota(jnp.int32, (QT, QT), 0) // BLK
        cb = lax.broadcasted_iota(jnp.int32, (QT, QT), 1) // BLK
        diag_mask = rb >= cb
        for t in range(N_QT):
            q_t = jnp.dot(x_ref[t * QT:(t + 1) * QT, :], wq_ref[...],
                          preferred_element_type=jnp.float32
                          ).astype(jnp.bfloat16)
            for h in range(H_LOCAL):
                qh = q_t[:, h * DH:(h + 1) * DH]
                kd = k_ref[t * QT:(t + 1) * QT, h * DH:(h + 1) * DH]
                sd = lax.dot_general(qh, kd, (((1,), (1,)), ((), ())),
                                     preferred_element_type=jnp.float32)
                ed = jnp.where(diag_mask, jnp.exp(sd * SCALE), 0.0)
                vd = v_ref[t * QT:(t + 1) * QT, h * DH:(h + 1) * DH]
                if t > 0:
                    kf = k_ref[0:t * QT, h * DH:(h + 1) * DH]
                    sf = lax.dot_general(qh, kf, (((1,), (1,)), ((), ())),
                                         preferred_element_type=jnp.float32)
                    ef = jnp.exp(sf * SCALE)
                    d = (jnp.sum(ef, axis=1, keepdims=True)
                         + jnp.sum(ed, axis=1, keepdims=True))
                    ctx = (jnp.dot(ef, v_ref[0:t * QT, h * DH:(h + 1) * DH],
                                   preferred_element_type=jnp.float32)
                           + jnp.dot(ed, vd,
                                     preferred_element_type=jnp.float32))
                else:
                    d = jnp.sum(ed, axis=1, keepdims=True)
                    ctx = jnp.dot(ed, vd, preferred_element_type=jnp.float32)
                ctx_ref[:, h * DH:(h + 1) * DH] = (
                    (ctx * (1.0 / d)).astype(jnp.bfloat16))
            out_ref[t * QT:(t + 1) * QT, :] = jnp.dot(
                ctx_ref[...], wo_ref[...],
                preferred_element_type=jnp.float32)
            if not _NO_RING:
                for j in range(CPT):
                    c = t * CPT + j
                    sb_ref[chunk_sl(c)] = (
                        out_ref[chunk_sl(c)].astype(jnp.bfloat16))
                    rdma = rs1_rdma(c)

                    @pl.when(lax.rem(me, N_PLANE) != c % N_PLANE)
                    def _():
                        rdma.start()

        if not _NO_RING:
            for idx in range(12):
                pltpu.make_async_remote_copy(
                    src_ref=sb_ref.at[chunk_sl(0)],
                    dst_ref=rs1_ref.at[idx // 3, idx % 3],
                    send_sem=rs1_send.at[idx], recv_sem=rs1_recv.at[idx],
                    device_id=(0,), device_id_type=pl.DeviceIdType.MESH,
                ).wait_recv()
            for j in range(N_Z):
                cj = N_PLANE * j + p_me
                agg = out_ref[pl.ds(cj * CHUNK, CHUNK), :]
                for k in range(3):
                    agg = agg + rs1_ref[j, k].astype(jnp.float32)
                @pl.when(j == z_me)
                def _():
                    acc_ref[...] = agg
                sb_ref[pl.ds(cj * CHUNK, CHUNK), :] = agg.astype(jnp.bfloat16)
                dz = lax.rem(z_me - j + N_Z, N_Z)
                zslot = jnp.maximum(dz, 1) - 1
                rdma2 = pltpu.make_async_remote_copy(
                    src_ref=sb_ref.at[(pl.ds(cj * CHUNK, CHUNK),
                                       slice(None))],
                    dst_ref=rs2_ref.at[zslot],
                    send_sem=rs2_send.at[zslot], recv_sem=rs2_recv.at[zslot],
                    device_id=(N_PLANE * j + p_me,),
                    device_id_type=pl.DeviceIdType.MESH)

                @pl.when(j != z_me)
                def _():
                    rdma2.start()

            my_sl = (pl.ds(me * CHUNK, CHUNK), slice(None))
            dummy_sl = (pl.ds(0, CHUNK), slice(None))

            def dummy_rdma(send_sems, recv_sems, k):
                return pltpu.make_async_remote_copy(
                    src_ref=sb_ref.at[dummy_sl], dst_ref=rs2_ref.at[k % 3],
                    send_sem=send_sems.at[k], recv_sem=recv_sems.at[k],
                    device_id=(0,), device_id_type=pl.DeviceIdType.MESH)

            for k in range(3):
                dummy_rdma(rs2_send, rs2_recv, k).wait_recv()
            red = acc_ref[...]
            for k in range(3):
                red = red + rs2_ref[k].astype(jnp.float32)
            sb_ref[my_sl] = red.astype(jnp.bfloat16)

            for idx in range(12):
                pltpu.make_async_remote_copy(
                    src_ref=sb_ref.at[dummy_sl],
                    dst_ref=rs1_ref.at[idx // 3, idx % 3],
                    send_sem=rs1_send.at[idx], recv_sem=rs1_recv.at[idx],
                    device_id=(0,), device_id_type=pl.DeviceIdType.MESH,
                ).wait_send()
            for k in range(3):
                dummy_rdma(rs2_send, rs2_recv, k).wait_send()

            for dz in range(1, N_Z):
                pltpu.make_async_remote_copy(
                    src_ref=sb_ref.at[my_sl], dst_ref=sb_ref.at[my_sl],
                    send_sem=aga_send.at[dz - 1], recv_sem=aga_recv.at[dz - 1],
                    device_id=(z_peer(dz),),
                    device_id_type=pl.DeviceIdType.MESH,
                ).start()
            for k in range(3):
                pltpu.make_async_remote_copy(
                    src_ref=sb_ref.at[dummy_sl], dst_ref=sb_ref.at[dummy_sl],
                    send_sem=aga_send.at[k], recv_sem=aga_recv.at[k],
                    device_id=(0,), device_id_type=pl.DeviceIdType.MESH,
                ).wait_recv()

            for j in range(N_Z):
                cj_sl = (pl.ds((N_PLANE * j + p_me) * CHUNK, CHUNK),
                         slice(None))
                for dp in range(1, N_PLANE):
                    pltpu.make_async_remote_copy(
                        src_ref=sb_ref.at[cj_sl], dst_ref=sb_ref.at[cj_sl],
                        send_sem=agb_send.at[j * 3 + dp - 1],
                        recv_sem=agb_recv.at[j * 3 + dp - 1],
                        device_id=(plane_peer(dp),),
                        device_id_type=pl.DeviceIdType.MESH,
                    ).start()
            for idx in range(12):
                pltpu.make_async_remote_copy(
                    src_ref=sb_ref.at[dummy_sl], dst_ref=sb_ref.at[dummy_sl],
                    send_sem=agb_send.at[idx], recv_sem=agb_recv.at[idx],
                    device_id=(0,), device_id_type=pl.DeviceIdType.MESH,
                ).wait_recv()

            out_ref[...] = sb_ref[...].astype(jnp.float32)

            for k in range(3):
                pltpu.make_async_remote_copy(
                    src_ref=sb_ref.at[my_sl], dst_ref=sb_ref.at[my_sl],
                    send_sem=aga_send.at[k], recv_sem=aga_recv.at[k],
                    device_id=(0,), device_id_type=pl.DeviceIdType.MESH,
                ).wait_send()
            for idx in range(12):
                pltpu.make_async_remote_copy(
                    src_ref=sb_ref.at[dummy_sl], dst_ref=sb_ref.at[dummy_sl],
                    send_sem=agb_send.at[idx], recv_sem=agb_recv.at[idx],
                    device_id=(0,), device_id_type=pl.DeviceIdType.MESH,
                ).wait_send()

        def _exit(second_barrier):
            for peer in peers:
                pl.semaphore_signal(second_barrier, inc=1, device_id=(peer,),
                                    device_id_type=pl.DeviceIdType.MESH)
            pl.semaphore_wait(second_barrier, len(peers))
        pl.run_scoped(_exit, second_barrier=pltpu.SemaphoreType.REGULAR)

    out = pl.pallas_call(
        body,
        out_shape=jax.ShapeDtypeStruct((SQ, D_MODEL), jnp.float32),
        in_specs=[pl.BlockSpec(memory_space=pltpu.VMEM)] * 5,
        out_specs=pl.BlockSpec(memory_space=pltpu.VMEM),
        scratch_shapes=[
            pltpu.VMEM((N_Z, 3, CHUNK, D_MODEL), jnp.bfloat16),
            pltpu.VMEM((3, CHUNK, D_MODEL), jnp.bfloat16),
            pltpu.VMEM((SQ, D_MODEL), jnp.bfloat16),
            pltpu.VMEM((CHUNK, D_MODEL), jnp.float32),
            pltpu.VMEM((QT, HD_LOCAL), jnp.bfloat16),
            pltpu.SemaphoreType.DMA((12,)),
            pltpu.SemaphoreType.DMA((12,)),
            pltpu.SemaphoreType.DMA((3,)),
            pltpu.SemaphoreType.DMA((3,)),
            pltpu.SemaphoreType.DMA((3,)),
            pltpu.SemaphoreType.DMA((3,)),
            pltpu.SemaphoreType.DMA((12,)),
            pltpu.SemaphoreType.DMA((12,)),
        ],
        compiler_params=pltpu.CompilerParams(
            collective_id=0, vmem_limit_bytes=100 * 1024 * 1024),
    )(x2, wq_l, k2, v2, wo_l)
    return out.reshape(1, SQ, D_MODEL)
